# Initial kernel scaffold; baseline (speedup 1.0000x reference)
import jax
import jax.numpy as jnp
from jax.experimental import pallas as pl


def kernel(x, embedding, W, b):
    raise NotImplementedError("write your pallas kernel here")


# speedup vs baseline: 6.9088x; 6.9088x over previous
import functools

import jax
import jax.numpy as jnp
from jax import lax
from jax.experimental import pallas as pl
from jax.experimental.pallas import tpu as pltpu
from jax.experimental.pallas import tpu_sc as plsc

VOCAB = 1000000
EMBED = 16
BATCH = 16384
HIST = 200

TC_BLOCK = 8192
TC_GRID = 123
NPAD = TC_BLOCK * TC_GRID

NC = 2
NS = 16
NW = NC * NS
LANES = 16
ROWS_PER_W = BATCH // NW
GROUPS_PER_W = ROWS_PER_W // LANES
IDX_ROWS = (HIST * LANES) // 128


def _tc_compress_body(w_ref, b_ref, emb_ref, t_ref):
    e = emb_ref[...]
    w = w_ref[...]
    t_ref[...] = jnp.dot(e, w, preferred_element_type=jnp.float32) + b_ref[0]


def _tc_compress(emb, w, b):
    return pl.pallas_call(
        _tc_compress_body,
        grid=(TC_GRID,),
        in_specs=[
            pl.BlockSpec((EMBED, 1), lambda i: (0, 0)),
            pl.BlockSpec(memory_space=pltpu.SMEM),
            pl.BlockSpec((TC_BLOCK, EMBED), lambda i: (i, 0)),
        ],
        out_specs=pl.BlockSpec((TC_BLOCK, 1), lambda i: (i, 0)),
        out_shape=jax.ShapeDtypeStruct((NPAD, 1), jnp.float32),
    )(w, b, emb)


def _sc_pool_body(t_hbm, xt_hbm, out_hbm, t_spmem, idx_v, vals_v, outw, gsem):
    c = lax.axis_index("c")
    s = lax.axis_index("s")
    wid = s * NC + c

    @pl.when(s == 0)
    def _fill():
        pltpu.sync_copy(t_hbm, t_spmem)

    plsc.subcore_barrier()

    def group_body(g, carry):
        gid = wid * GROUPS_PER_W + g
        pltpu.sync_copy(xt_hbm.at[gid], idx_v)
        copies = []
        for r in range(IDX_ROWS):
            copies.append(
                pltpu.async_copy(t_spmem.at[idx_v.at[r]], vals_v.at[r], gsem)
            )
        for cp in copies:
            cp.wait()
        acc = jnp.zeros((LANES,), jnp.float32)
        for r in range(IDX_ROWS):
            for q in range(128 // LANES):
                acc = acc + vals_v[r, pl.ds(q * LANES, LANES)]
        out16 = 1.0 / (1.0 + jnp.exp(-acc))
        outw[pl.ds(g * LANES, LANES)] = out16
        return carry

    lax.fori_loop(0, GROUPS_PER_W, group_body, 0)
    pltpu.sync_copy(outw, out_hbm.at[pl.ds(wid * ROWS_PER_W, ROWS_PER_W)])


_sc_pool = functools.partial(
    pl.kernel,
    out_type=jax.ShapeDtypeStruct((BATCH,), jnp.float32),
    mesh=plsc.VectorSubcoreMesh(core_axis_name="c", subcore_axis_name="s"),
    scratch_types=[
        pltpu.VMEM_SHARED((NPAD,), jnp.float32),
        pltpu.VMEM((IDX_ROWS, 128), jnp.int32),
        pltpu.VMEM((IDX_ROWS, 128), jnp.float32),
        pltpu.VMEM((ROWS_PER_W,), jnp.float32),
        pltpu.SemaphoreType.DMA,
    ],
)(_sc_pool_body)


def kernel(x, embedding, W, b):
    x = x.astype(jnp.int32)
    t = _tc_compress(embedding, W * (1.0 / HIST), b * (1.0 / HIST)).reshape(NPAD)
    xt = (
        x.reshape(BATCH // LANES, LANES, HIST)
        .transpose(0, 2, 1)
        .reshape(BATCH // LANES, IDX_ROWS, 128)
    )
    out = _sc_pool(t, xt)
    return out.reshape(BATCH, 1)


# speedup vs baseline: 8.3965x; 1.2153x over previous
import functools

import jax
import jax.numpy as jnp
from jax import lax
from jax.experimental import pallas as pl
from jax.experimental.pallas import tpu as pltpu
from jax.experimental.pallas import tpu_sc as plsc

VOCAB = 1000000
EMBED = 16
BATCH = 16384
HIST = 200

TC_ROWS = VOCAB // 8
TC_BLOCK = 1000
TC_GRID = TC_ROWS // TC_BLOCK
NPAD = VOCAB

NC = 2
NS = 16
NW = NC * NS
LANES = 16
ROWS_PER_W = BATCH // NW
GROUPS_PER_W = ROWS_PER_W // LANES
IDX_ROWS = (HIST * LANES) // 128


def _tc_compress_body(w_ref, b_ref, emb_ref, t_ref):
    e = emb_ref[...]
    w = w_ref[...]
    t_ref[...] = jnp.dot(e, w, preferred_element_type=jnp.float32) + b_ref[0]


def _tc_compress(emb128, wk, b):
    return pl.pallas_call(
        _tc_compress_body,
        grid=(TC_GRID,),
        in_specs=[
            pl.BlockSpec((128, 8), lambda i: (0, 0)),
            pl.BlockSpec(memory_space=pltpu.SMEM),
            pl.BlockSpec((TC_BLOCK, 128), lambda i: (i, 0)),
        ],
        out_specs=pl.BlockSpec((TC_BLOCK, 8), lambda i: (i, 0)),
        out_shape=jax.ShapeDtypeStruct((TC_ROWS, 8), jnp.float32),
    )(wk, b, emb128)


def _sc_pool_body(t_hbm, xt_hbm, out_hbm, t_spmem, idx_v, vals_v, outw, gsem):
    c = lax.axis_index("c")
    s = lax.axis_index("s")
    wid = s * NC + c

    @pl.when(s == 0)
    def _fill():
        pltpu.sync_copy(t_hbm, t_spmem)

    plsc.subcore_barrier()

    def group_body(g, carry):
        gid = wid * GROUPS_PER_W + g
        pltpu.sync_copy(xt_hbm.at[gid], idx_v)
        copies = []
        for r in range(IDX_ROWS):
            copies.append(
                pltpu.async_copy(t_spmem.at[idx_v.at[r]], vals_v.at[r], gsem)
            )
        for cp in copies:
            cp.wait()
        acc = jnp.zeros((LANES,), jnp.float32)
        for r in range(IDX_ROWS):
            for q in range(128 // LANES):
                acc = acc + vals_v[r, pl.ds(q * LANES, LANES)]
        out16 = 1.0 / (1.0 + jnp.exp(-acc))
        outw[pl.ds(g * LANES, LANES)] = out16
        return carry

    lax.fori_loop(0, GROUPS_PER_W, group_body, 0)
    pltpu.sync_copy(outw, out_hbm.at[pl.ds(wid * ROWS_PER_W, ROWS_PER_W)])


_sc_pool = functools.partial(
    pl.kernel,
    out_type=jax.ShapeDtypeStruct((BATCH,), jnp.float32),
    mesh=plsc.VectorSubcoreMesh(core_axis_name="c", subcore_axis_name="s"),
    scratch_types=[
        pltpu.VMEM_SHARED((NPAD,), jnp.float32),
        pltpu.VMEM((IDX_ROWS, 128), jnp.int32),
        pltpu.VMEM((IDX_ROWS, 128), jnp.float32),
        pltpu.VMEM((ROWS_PER_W,), jnp.float32),
        pltpu.SemaphoreType.DMA,
    ],
)(_sc_pool_body)


def kernel(x, embedding, W, b):
    x = x.astype(jnp.int32)
    wk = jnp.kron(jnp.eye(8, dtype=jnp.float32), W * (1.0 / HIST))
    t = _tc_compress(
        embedding.reshape(TC_ROWS, 128), wk, b * (1.0 / HIST)
    ).reshape(NPAD)
    xt = (
        x.reshape(BATCH // LANES, LANES, HIST)
        .transpose(0, 2, 1)
        .reshape(BATCH // LANES, IDX_ROWS, 128)
    )
    out = _sc_pool(t, xt)
    return out.reshape(BATCH, 1)


# speedup vs baseline: 38.1435x; 4.5428x over previous
import functools

import jax
import jax.numpy as jnp
from jax import lax
from jax.experimental import pallas as pl
from jax.experimental.pallas import tpu as pltpu
from jax.experimental.pallas import tpu_sc as plsc

VOCAB = 1000000
EMBED = 16
BATCH = 16384
HIST = 200

TC_BN = 65536
TC_GRID = 16
NPAD = TC_BN * TC_GRID

NC = 2
NS = 16
NW = NC * NS
LANES = 16
ROWS_PER_W = BATCH // NW
GROUPS_PER_W = ROWS_PER_W // LANES
IDX_ROWS = (HIST * LANES) // 128


def _tc_compress_body(w_ref, b_ref, embt_ref, t_ref):
    e = embt_ref[...]
    w = w_ref[...]
    t = jnp.dot(w, e, preferred_element_type=jnp.float32) + b_ref[0]
    t_ref[...] = t.reshape(TC_BN)


def _tc_compress(embt, wt, b):
    return pl.pallas_call(
        _tc_compress_body,
        grid=(TC_GRID,),
        in_specs=[
            pl.BlockSpec((1, EMBED), lambda i: (0, 0)),
            pl.BlockSpec(memory_space=pltpu.SMEM),
            pl.BlockSpec((EMBED, TC_BN), lambda i: (0, i)),
        ],
        out_specs=pl.BlockSpec((TC_BN,), lambda i: (i,)),
        out_shape=jax.ShapeDtypeStruct((NPAD,), jnp.float32),
    )(wt, b, embt)


def _sc_pool_body(t_hbm, xt_hbm, out_hbm, t_spmem, idx_v, vals_v, outw, gsem):
    c = lax.axis_index("c")
    s = lax.axis_index("s")
    wid = s * NC + c

    @pl.when(s == 0)
    def _fill():
        pltpu.sync_copy(t_hbm, t_spmem)

    plsc.subcore_barrier()

    def group_body(g, carry):
        gid = wid * GROUPS_PER_W + g
        pltpu.sync_copy(xt_hbm.at[gid], idx_v)
        copies = []
        for r in range(IDX_ROWS):
            copies.append(
                pltpu.async_copy(t_spmem.at[idx_v.at[r]], vals_v.at[r], gsem)
            )
        for cp in copies:
            cp.wait()
        acc = jnp.zeros((LANES,), jnp.float32)
        for r in range(IDX_ROWS):
            for q in range(128 // LANES):
                acc = acc + vals_v[r, pl.ds(q * LANES, LANES)]
        out16 = 1.0 / (1.0 + jnp.exp(-acc))
        outw[pl.ds(g * LANES, LANES)] = out16
        return carry

    lax.fori_loop(0, GROUPS_PER_W, group_body, 0)
    pltpu.sync_copy(outw, out_hbm.at[pl.ds(wid * ROWS_PER_W, ROWS_PER_W)])


_sc_pool = functools.partial(
    pl.kernel,
    out_type=jax.ShapeDtypeStruct((BATCH,), jnp.float32),
    mesh=plsc.VectorSubcoreMesh(core_axis_name="c", subcore_axis_name="s"),
    scratch_types=[
        pltpu.VMEM_SHARED((NPAD,), jnp.float32),
        pltpu.VMEM((IDX_ROWS, 128), jnp.int32),
        pltpu.VMEM((IDX_ROWS, 128), jnp.float32),
        pltpu.VMEM((ROWS_PER_W,), jnp.float32),
        pltpu.SemaphoreType.DMA,
    ],
)(_sc_pool_body)


def kernel(x, embedding, W, b):
    x = x.astype(jnp.int32)
    t = _tc_compress(embedding.T, W.reshape(1, EMBED) * (1.0 / HIST),
                     b * (1.0 / HIST))
    xt = (
        x.reshape(BATCH // LANES, LANES, HIST)
        .transpose(0, 2, 1)
        .reshape(BATCH // LANES, IDX_ROWS, 128)
    )
    out = _sc_pool(t, xt)
    return out.reshape(BATCH, 1)


# speedup vs baseline: 38.2253x; 1.0021x over previous
import functools

import jax
import jax.numpy as jnp
from jax import lax
from jax.experimental import pallas as pl
from jax.experimental.pallas import tpu as pltpu
from jax.experimental.pallas import tpu_sc as plsc

VOCAB = 1000000
EMBED = 16
BATCH = 16384
HIST = 200

TC_BN = 65536
TC_GRID = 16
NPAD = TC_BN * TC_GRID

NC = 2
NS = 16
NW = NC * NS
LANES = 16
ROWS_PER_W = BATCH // NW
GROUPS_PER_W = ROWS_PER_W // LANES
IDX_ROWS = (HIST * LANES) // 128


def _tc_compress_body(w_ref, b_ref, embt_ref, t_ref):
    e = embt_ref[...]
    w = w_ref[...]
    t = jnp.dot(w, e, preferred_element_type=jnp.float32) + b_ref[0]
    t_ref[...] = t.reshape(TC_BN)


def _tc_compress(embt, wt, b):
    return pl.pallas_call(
        _tc_compress_body,
        grid=(TC_GRID,),
        in_specs=[
            pl.BlockSpec((1, EMBED), lambda i: (0, 0)),
            pl.BlockSpec(memory_space=pltpu.SMEM),
            pl.BlockSpec((EMBED, TC_BN), lambda i: (0, i)),
        ],
        out_specs=pl.BlockSpec((TC_BN,), lambda i: (i,)),
        out_shape=jax.ShapeDtypeStruct((NPAD,), jnp.float32),
    )(wt, b, embt)


def _sc_pool_body(t_hbm, xt_hbm, out_hbm, t_spmem, idx_v, vals_v, outw, gsem):
    c = lax.axis_index("c")
    s = lax.axis_index("s")
    wid = s * NC + c

    @pl.when(s == 0)
    def _fill():
        pltpu.sync_copy(t_hbm, t_spmem)

    plsc.subcore_barrier()

    def group_body(g, carry):
        gid = wid * GROUPS_PER_W + g
        pltpu.sync_copy(xt_hbm.at[gid], idx_v)
        pltpu.async_copy(t_spmem.at[idx_v], vals_v, gsem).wait()
        acc = jnp.zeros((LANES,), jnp.float32)
        for j in range(HIST):
            acc = acc + vals_v[pl.ds(j * LANES, LANES)]
        out16 = 1.0 / (1.0 + jnp.exp(-acc))
        outw[pl.ds(g * LANES, LANES)] = out16
        return carry

    lax.fori_loop(0, GROUPS_PER_W, group_body, 0)
    pltpu.sync_copy(outw, out_hbm.at[pl.ds(wid * ROWS_PER_W, ROWS_PER_W)])


_sc_pool = functools.partial(
    pl.kernel,
    out_type=jax.ShapeDtypeStruct((BATCH,), jnp.float32),
    mesh=plsc.VectorSubcoreMesh(core_axis_name="c", subcore_axis_name="s"),
    scratch_types=[
        pltpu.VMEM_SHARED((NPAD,), jnp.float32),
        pltpu.VMEM((HIST * LANES,), jnp.int32),
        pltpu.VMEM((HIST * LANES,), jnp.float32),
        pltpu.VMEM((ROWS_PER_W,), jnp.float32),
        pltpu.SemaphoreType.DMA,
    ],
)(_sc_pool_body)


def kernel(x, embedding, W, b):
    x = x.astype(jnp.int32)
    t = _tc_compress(embedding.T, W.reshape(1, EMBED) * (1.0 / HIST),
                     b * (1.0 / HIST))
    xt = (
        x.reshape(BATCH // LANES, LANES, HIST)
        .transpose(0, 2, 1)
        .reshape(BATCH // LANES, HIST * LANES)
    )
    out = _sc_pool(t, xt)
    return out.reshape(BATCH, 1)
